# Initial kernel scaffold; baseline (speedup 1.0000x reference)
import functools

import jax
import jax.numpy as jnp
from jax import lax
from jax.experimental import pallas as pl
from jax.experimental.pallas import tpu as pltpu
from jax.experimental.pallas import tpu_sc as plsc

N_TOK, N_ATOM = 4096, 32768
D_FEAT, D_SINGLE, N_RESTYPE = 128, 384, 32

NC, NS = 2, 16
NW = NC * NS
ATOMS_PER_W = N_ATOM // NW
CHUNK = 128
NCHUNK = ATOMS_PER_W // CHUNK
TOK_STRIPE = N_TOK // NS
CNT_W = 16

TOK_BLK = 1024


def _seg_sum_body(feats_hbm, idx_hbm, zsum_hbm, zcnt_hbm, ones_hbm,
                  out_sum_hbm, out_cnt_hbm,
                  shared_sum, shared_cnt, idx_v, rows_v, ones_v):
    c = lax.axis_index("c")
    s = lax.axis_index("s")
    wid = c * NS + s

    tok0 = s * TOK_STRIPE
    pltpu.sync_copy(zsum_hbm.at[pl.ds(tok0, TOK_STRIPE)],
                    shared_sum.at[pl.ds(tok0, TOK_STRIPE)])
    pltpu.sync_copy(zcnt_hbm.at[pl.ds(tok0, TOK_STRIPE)],
                    shared_cnt.at[pl.ds(tok0, TOK_STRIPE)])
    pltpu.sync_copy(ones_hbm, ones_v)
    pltpu.sync_copy(idx_hbm.at[pl.ds(wid * NCHUNK, NCHUNK)], idx_v)
    plsc.subcore_barrier()

    a0 = wid * ATOMS_PER_W
    for j in range(NCHUNK):
        pltpu.sync_copy(feats_hbm.at[pl.ds(a0 + j * CHUNK, CHUNK)], rows_v)
        pltpu.sync_copy(rows_v, shared_sum.at[idx_v.at[j]], add=True)
        pltpu.sync_copy(ones_v, shared_cnt.at[idx_v.at[j]], add=True)
    plsc.subcore_barrier()

    pltpu.sync_copy(shared_sum.at[pl.ds(tok0, TOK_STRIPE)],
                    out_sum_hbm.at[c, pl.ds(tok0, TOK_STRIPE)])
    pltpu.sync_copy(shared_cnt.at[pl.ds(tok0, TOK_STRIPE)],
                    out_cnt_hbm.at[c, pl.ds(tok0, TOK_STRIPE)])


def _segment_sum_sc(feats, idx):
    mesh = plsc.VectorSubcoreMesh(core_axis_name="c", subcore_axis_name="s")
    kern = pl.kernel(
        _seg_sum_body,
        out_type=[
            jax.ShapeDtypeStruct((NC, N_TOK, D_FEAT), jnp.float32),
            jax.ShapeDtypeStruct((NC, N_TOK, CNT_W), jnp.float32),
        ],
        mesh=mesh,
        scratch_types=[
            pltpu.VMEM_SHARED((N_TOK, D_FEAT), jnp.float32),
            pltpu.VMEM_SHARED((N_TOK, CNT_W), jnp.float32),
            pltpu.VMEM((NCHUNK, CHUNK), jnp.int32),
            pltpu.VMEM((CHUNK, D_FEAT), jnp.float32),
            pltpu.VMEM((CHUNK, CNT_W), jnp.float32),
        ],
    )
    zsum = jnp.zeros((N_TOK, D_FEAT), jnp.float32)
    zcnt = jnp.zeros((N_TOK, CNT_W), jnp.float32)
    ones = jnp.ones((CHUNK, CNT_W), jnp.float32)
    idx2d = idx.reshape(NW * NCHUNK, CHUNK)
    return kern(feats, idx2d, zsum, zcnt, ones)


def _combine_body(ps_ref, pc_ref, rt_ref, table_ref, wa_ref, wp_ref, out_ref):
    seg_sum = ps_ref[0] + ps_ref[1]
    cnt = pc_ref[0, :, 0:1] + pc_ref[1, :, 0:1]
    mean = seg_sum / jnp.maximum(cnt, 1.0)
    w = jnp.dot(wa_ref[...], wp_ref[...], preferred_element_type=jnp.float32)
    rt = rt_ref[0]
    onehot = (lax.broadcasted_iota(jnp.int32, (N_RESTYPE, TOK_BLK), 0)
              == rt).astype(jnp.float32)
    emb = lax.dot_general(onehot, table_ref[...],
                          dimension_numbers=(((0,), (0,)), ((), ())),
                          preferred_element_type=jnp.float32)
    out_ref[...] = emb + jnp.dot(mean, w, preferred_element_type=jnp.float32)


def _combine_tc(psum, pcnt, restype, table, w_atom, w_proj):
    n_blk = N_TOK // TOK_BLK
    rt3 = restype.reshape(n_blk, 1, TOK_BLK)
    return pl.pallas_call(
        _combine_body,
        grid=(n_blk,),
        in_specs=[
            pl.BlockSpec((NC, TOK_BLK, D_FEAT), lambda i: (0, i, 0)),
            pl.BlockSpec((NC, TOK_BLK, CNT_W), lambda i: (0, i, 0)),
            pl.BlockSpec((1, 1, TOK_BLK), lambda i: (i, 0, 0)),
            pl.BlockSpec((N_RESTYPE, D_SINGLE), lambda i: (0, 0)),
            pl.BlockSpec((D_FEAT, D_FEAT), lambda i: (0, 0)),
            pl.BlockSpec((D_FEAT, D_SINGLE), lambda i: (0, 0)),
        ],
        out_specs=pl.BlockSpec((TOK_BLK, D_SINGLE), lambda i: (i, 0)),
        out_shape=jax.ShapeDtypeStruct((N_TOK, D_SINGLE), jnp.float32),
    )(psum, pcnt, rt3, table, w_atom, w_proj)


def kernel(atom_feats, restype, atom_to_token_idx, restype_table, W_atom, W_proj):
    feats = atom_feats.reshape(N_ATOM, D_FEAT)
    idx = atom_to_token_idx.reshape(N_ATOM).astype(jnp.int32)
    rt = restype.reshape(N_TOK).astype(jnp.int32)
    psum, pcnt = _segment_sum_sc(feats, idx)
    out = _combine_tc(psum, pcnt, rt, restype_table, W_atom, W_proj)
    return out.reshape(1, N_TOK, D_SINGLE)


# speedup vs baseline: 3.0459x; 3.0459x over previous
import functools

import jax
import jax.numpy as jnp
from jax import lax
from jax.experimental import pallas as pl
from jax.experimental.pallas import tpu as pltpu
from jax.experimental.pallas import tpu_sc as plsc

N_TOK, N_ATOM = 4096, 32768
D_FEAT, D_SINGLE, N_RESTYPE = 128, 384, 32

NC, NS = 2, 16
NW = NC * NS
ATOMS_PER_W = N_ATOM // NW
CHUNK = 128
NCHUNK = ATOMS_PER_W // CHUNK
TOK_STRIPE = N_TOK // NS
CNT_W = 128

TOK_BLK = 1024


def _seg_sum_body(feats_hbm, idx_hbm, zsum_hbm, zcnt_hbm, ones_hbm,
                  out_sum_hbm, out_cnt_hbm,
                  shared_sum, shared_cnt, idx_v, rows_v, ones_v):
    c = lax.axis_index("c")
    s = lax.axis_index("s")
    wid = c * NS + s

    tok0 = s * TOK_STRIPE
    pltpu.sync_copy(zsum_hbm.at[pl.ds(tok0, TOK_STRIPE)],
                    shared_sum.at[pl.ds(tok0, TOK_STRIPE)])
    pltpu.sync_copy(zcnt_hbm.at[pl.ds(tok0, TOK_STRIPE)],
                    shared_cnt.at[pl.ds(tok0, TOK_STRIPE)])
    pltpu.sync_copy(ones_hbm, ones_v)
    pltpu.sync_copy(idx_hbm.at[pl.ds(wid * NCHUNK, NCHUNK)], idx_v)
    plsc.subcore_barrier()

    a0 = wid * ATOMS_PER_W
    for j in range(NCHUNK):
        pltpu.sync_copy(feats_hbm.at[pl.ds(a0 + j * CHUNK, CHUNK)], rows_v)
        pltpu.sync_copy(rows_v, shared_sum.at[idx_v.at[j]], add=True)
        pltpu.sync_copy(ones_v, shared_cnt.at[idx_v.at[j]], add=True)
    plsc.subcore_barrier()

    pltpu.sync_copy(shared_sum.at[pl.ds(tok0, TOK_STRIPE)],
                    out_sum_hbm.at[c, pl.ds(tok0, TOK_STRIPE)])
    pltpu.sync_copy(shared_cnt.at[pl.ds(tok0, TOK_STRIPE)],
                    out_cnt_hbm.at[c, pl.ds(tok0, TOK_STRIPE)])


def _segment_sum_sc(feats, idx):
    mesh = plsc.VectorSubcoreMesh(core_axis_name="c", subcore_axis_name="s")
    kern = pl.kernel(
        _seg_sum_body,
        out_type=[
            jax.ShapeDtypeStruct((NC, N_TOK, D_FEAT), jnp.float32),
            jax.ShapeDtypeStruct((NC, N_TOK, CNT_W), jnp.float32),
        ],
        mesh=mesh,
        scratch_types=[
            pltpu.VMEM_SHARED((N_TOK, D_FEAT), jnp.float32),
            pltpu.VMEM_SHARED((N_TOK, CNT_W), jnp.float32),
            pltpu.VMEM((NCHUNK, CHUNK), jnp.int32),
            pltpu.VMEM((CHUNK, D_FEAT), jnp.float32),
            pltpu.VMEM((CHUNK, CNT_W), jnp.float32),
        ],
    )
    zsum = jnp.zeros((N_TOK, D_FEAT), jnp.float32)
    zcnt = jnp.zeros((N_TOK, CNT_W), jnp.float32)
    ones = jnp.ones((CHUNK, CNT_W), jnp.float32)
    idx2d = idx.reshape(NW * NCHUNK, CHUNK)
    return kern(feats, idx2d, zsum, zcnt, ones)


def _combine_body(ps_ref, pc_ref, rt_ref, table_ref, wa_ref, wp_ref, out_ref):
    seg_sum = ps_ref[0] + ps_ref[1]
    cnt = pc_ref[0, :, 0:1] + pc_ref[1, :, 0:1]
    mean = seg_sum / jnp.maximum(cnt, 1.0)
    w = jnp.dot(wa_ref[...], wp_ref[...], preferred_element_type=jnp.float32,
                precision=lax.Precision.HIGHEST)
    rt = rt_ref[0]
    onehot = (lax.broadcasted_iota(jnp.int32, (N_RESTYPE, TOK_BLK), 0)
              == rt).astype(jnp.float32)
    emb = lax.dot_general(onehot, table_ref[...],
                          dimension_numbers=(((0,), (0,)), ((), ())),
                          preferred_element_type=jnp.float32,
                          precision=lax.Precision.HIGHEST)
    out_ref[...] = emb + jnp.dot(mean, w, preferred_element_type=jnp.float32,
                                 precision=lax.Precision.HIGHEST)


def _combine_tc(psum, pcnt, restype, table, w_atom, w_proj):
    n_blk = N_TOK // TOK_BLK
    rt3 = restype.reshape(n_blk, 1, TOK_BLK)
    return pl.pallas_call(
        _combine_body,
        grid=(n_blk,),
        in_specs=[
            pl.BlockSpec((NC, TOK_BLK, D_FEAT), lambda i: (0, i, 0)),
            pl.BlockSpec((NC, TOK_BLK, CNT_W), lambda i: (0, i, 0)),
            pl.BlockSpec((1, 1, TOK_BLK), lambda i: (i, 0, 0)),
            pl.BlockSpec((N_RESTYPE, D_SINGLE), lambda i: (0, 0)),
            pl.BlockSpec((D_FEAT, D_FEAT), lambda i: (0, 0)),
            pl.BlockSpec((D_FEAT, D_SINGLE), lambda i: (0, 0)),
        ],
        out_specs=pl.BlockSpec((TOK_BLK, D_SINGLE), lambda i: (i, 0)),
        out_shape=jax.ShapeDtypeStruct((N_TOK, D_SINGLE), jnp.float32),
    )(psum, pcnt, rt3, table, w_atom, w_proj)


def kernel(atom_feats, restype, atom_to_token_idx, restype_table, W_atom, W_proj):
    feats = atom_feats.reshape(N_ATOM, D_FEAT)
    idx = atom_to_token_idx.reshape(N_ATOM).astype(jnp.int32)
    rt = restype.reshape(N_TOK).astype(jnp.int32)
    psum, pcnt = _segment_sum_sc(feats, idx)
    out = _combine_tc(psum, pcnt, rt, restype_table, W_atom, W_proj)
    return out.reshape(1, N_TOK, D_SINGLE)


# speedup vs baseline: 3.1006x; 1.0180x over previous
import functools

import jax
import jax.numpy as jnp
from jax import lax
from jax.experimental import pallas as pl
from jax.experimental.pallas import tpu as pltpu
from jax.experimental.pallas import tpu_sc as plsc

N_TOK, N_ATOM = 4096, 32768
D_FEAT, D_SINGLE, N_RESTYPE = 128, 384, 32

NC, NS = 2, 16
NW = NC * NS
ATOMS_PER_W = N_ATOM // NW
CHUNK = 128
NCHUNK = ATOMS_PER_W // CHUNK
TOK_STRIPE = N_TOK // NS
BROWS = N_TOK // 128

TOK_BLK = 1024
_END_SENTINEL = 1 << 30


def _seg_sum_body(feats_hbm, idx_hbm, idxflat_hbm, out_sum_hbm, out_st_hbm,
                  out_en_hbm,
                  shared_sum, shared_st, shared_en,
                  idx2d_v, idxb_v, rows_a, rows_b, st_loc, en_loc,
                  zf_v, zi_v, ident_v, sem_a, sem_b, sem_sc):
    c = lax.axis_index("c")
    s = lax.axis_index("s")
    wid = c * NS + s
    a0 = wid * ATOMS_PER_W
    tok0 = s * TOK_STRIPE

    zf16 = jnp.zeros((16,), jnp.float32)
    zi16 = jnp.zeros((16,), jnp.int32)
    for r in range(16):
        for cc in range(8):
            zf_v[r, pl.ds(cc * 16, 16)] = zf16
            zi_v[r, pl.ds(cc * 16, 16)] = zi16
    i16 = lax.iota(jnp.int32, 16)
    ident_v[pl.ds(0, 16)] = i16
    ident_v[pl.ds(16, 16)] = i16 + 16

    for r in range(BROWS):
        for cc in range(8):
            st_loc[r, pl.ds(cc * 16, 16)] = zi16
            en_loc[r, pl.ds(cc * 16, 16)] = zi16
    for k in range(TOK_STRIPE // 16):
        pltpu.sync_copy(zf_v, shared_sum.at[pl.ds(tok0 + 16 * k, 16)])
    pltpu.sync_copy(zi_v.at[pl.ds(0, 2)], shared_st.at[pl.ds(2 * s, 2)])
    pltpu.sync_copy(zi_v.at[pl.ds(0, 2)], shared_en.at[pl.ds(2 * s, 2)])

    pltpu.sync_copy(idx_hbm.at[pl.ds(wid * NCHUNK, NCHUNK)], idx2d_v)
    pltpu.sync_copy(idxflat_hbm.at[pl.ds(a0, ATOMS_PER_W + 16)], idxb_v)

    plsc.subcore_barrier()

    bufs = (rows_a, rows_b)
    sems = (sem_a, sem_b)

    def _stage(j):
        return pltpu.async_copy(
            feats_hbm.at[pl.ds(a0 + j * CHUNK, CHUNK)], bufs[j % 2], sems[j % 2])

    stage_descs = {0: _stage(0)}
    for j in range(NCHUNK):
        if j + 1 < NCHUNK:
            stage_descs[j + 1] = _stage(j + 1)
        stage_descs.pop(j).wait()
        scat = pltpu.async_copy(
            bufs[j % 2], shared_sum.at[idx2d_v.at[j]], sem_sc, add=True)
        for k in range(CHUNK // 16):
            base = 8 + j * CHUNK + 16 * k
            x = plsc.load_gather(idxb_v, [base + i16])
            xp = plsc.load_gather(idxb_v, [base - 1 + i16])
            xn = plsc.load_gather(idxb_v, [base + 1 + i16])
            gpos = (a0 + j * CHUNK + 16 * k) + i16
            row = lax.shift_right_logical(x, 7)
            col = lax.bitwise_and(x, 127)
            plsc.store_scatter(st_loc, [row, col], gpos, mask=x != xp)
            plsc.store_scatter(en_loc, [row, col], gpos + 1, mask=x != xn)
        scat.wait()
    plsc.subcore_barrier()

    pltpu.sync_copy(st_loc, shared_st.at[ident_v], add=True)
    pltpu.sync_copy(en_loc, shared_en.at[ident_v], add=True)
    plsc.subcore_barrier()

    pltpu.sync_copy(shared_sum.at[pl.ds(tok0, TOK_STRIPE)],
                    out_sum_hbm.at[c, pl.ds(tok0, TOK_STRIPE)])
    pltpu.sync_copy(shared_st.at[pl.ds(2 * s, 2)], out_st_hbm.at[c, pl.ds(2 * s, 2)])
    pltpu.sync_copy(shared_en.at[pl.ds(2 * s, 2)], out_en_hbm.at[c, pl.ds(2 * s, 2)])


def _segment_sum_sc(feats, idx):
    mesh = plsc.VectorSubcoreMesh(core_axis_name="c", subcore_axis_name="s")
    kern = pl.kernel(
        _seg_sum_body,
        out_type=[
            jax.ShapeDtypeStruct((NC, N_TOK, D_FEAT), jnp.float32),
            jax.ShapeDtypeStruct((NC, BROWS, 128), jnp.int32),
            jax.ShapeDtypeStruct((NC, BROWS, 128), jnp.int32),
        ],
        mesh=mesh,
        scratch_types=[
            pltpu.VMEM_SHARED((N_TOK, D_FEAT), jnp.float32),
            pltpu.VMEM_SHARED((BROWS, 128), jnp.int32),
            pltpu.VMEM_SHARED((BROWS, 128), jnp.int32),
            pltpu.VMEM((NCHUNK, CHUNK), jnp.int32),
            pltpu.VMEM((ATOMS_PER_W + 16,), jnp.int32),
            pltpu.VMEM((CHUNK, D_FEAT), jnp.float32),
            pltpu.VMEM((CHUNK, D_FEAT), jnp.float32),
            pltpu.VMEM((BROWS, 128), jnp.int32),
            pltpu.VMEM((BROWS, 128), jnp.int32),
            pltpu.VMEM((16, 128), jnp.float32),
            pltpu.VMEM((16, 128), jnp.int32),
            pltpu.VMEM((32,), jnp.int32),
            pltpu.SemaphoreType.DMA,
            pltpu.SemaphoreType.DMA,
            pltpu.SemaphoreType.DMA,
        ],
        compiler_params=pltpu.CompilerParams(needs_layout_passes=False),
    )
    idx2d = idx.reshape(NW * NCHUNK, CHUNK)
    idxflat_padded = jnp.concatenate([
        jnp.full((8,), -1, jnp.int32), idx,
        jnp.full((8,), _END_SENTINEL, jnp.int32)])
    return kern(feats, idx2d, idxflat_padded)


def _combine_body(ps_ref, st_ref, en_ref, rt_ref, table_ref, wa_ref, wp_ref,
                  out_ref):
    seg_sum = ps_ref[0] + ps_ref[1]
    cnt = ((en_ref[0] + en_ref[1]) - (st_ref[0] + st_ref[1])
           ).astype(jnp.float32)
    mean = seg_sum / jnp.maximum(cnt, 1.0)
    w = jnp.dot(wa_ref[...], wp_ref[...], preferred_element_type=jnp.float32,
                precision=lax.Precision.HIGHEST)
    rt = rt_ref[0]
    onehot = (lax.broadcasted_iota(jnp.int32, (N_RESTYPE, TOK_BLK), 0)
              == rt).astype(jnp.float32)
    emb = lax.dot_general(onehot, table_ref[...],
                          dimension_numbers=(((0,), (0,)), ((), ())),
                          preferred_element_type=jnp.float32,
                          precision=lax.Precision.HIGHEST)
    out_ref[...] = emb + jnp.dot(mean, w, preferred_element_type=jnp.float32,
                                 precision=lax.Precision.HIGHEST)


def _combine_tc(psum, starts, ends, restype, table, w_atom, w_proj):
    n_blk = N_TOK // TOK_BLK
    rt3 = restype.reshape(n_blk, 1, TOK_BLK)
    st3 = starts.reshape(NC, N_TOK, 1)
    en3 = ends.reshape(NC, N_TOK, 1)
    return pl.pallas_call(
        _combine_body,
        grid=(n_blk,),
        in_specs=[
            pl.BlockSpec((NC, TOK_BLK, D_FEAT), lambda i: (0, i, 0)),
            pl.BlockSpec((NC, TOK_BLK, 1), lambda i: (0, i, 0)),
            pl.BlockSpec((NC, TOK_BLK, 1), lambda i: (0, i, 0)),
            pl.BlockSpec((1, 1, TOK_BLK), lambda i: (i, 0, 0)),
            pl.BlockSpec((N_RESTYPE, D_SINGLE), lambda i: (0, 0)),
            pl.BlockSpec((D_FEAT, D_FEAT), lambda i: (0, 0)),
            pl.BlockSpec((D_FEAT, D_SINGLE), lambda i: (0, 0)),
        ],
        out_specs=pl.BlockSpec((TOK_BLK, D_SINGLE), lambda i: (i, 0)),
        out_shape=jax.ShapeDtypeStruct((N_TOK, D_SINGLE), jnp.float32),
    )(psum, st3, en3, rt3, table, w_atom, w_proj)


def kernel(atom_feats, restype, atom_to_token_idx, restype_table, W_atom, W_proj):
    feats = atom_feats.reshape(N_ATOM, D_FEAT)
    idx = atom_to_token_idx.reshape(N_ATOM).astype(jnp.int32)
    rt = restype.reshape(N_TOK).astype(jnp.int32)
    psum, starts, ends = _segment_sum_sc(feats, idx)
    out = _combine_tc(psum, starts, ends, rt, restype_table, W_atom, W_proj)
    return out.reshape(1, N_TOK, D_SINGLE)


# speedup vs baseline: 3.7184x; 1.1993x over previous
import functools

import jax
import jax.numpy as jnp
from jax import lax
from jax.experimental import pallas as pl
from jax.experimental.pallas import tpu as pltpu
from jax.experimental.pallas import tpu_sc as plsc

N_TOK, N_ATOM = 4096, 32768
D_FEAT, D_SINGLE, N_RESTYPE = 128, 384, 32

NC, NS = 2, 16
NW = NC * NS
ATOMS_PER_W = N_ATOM // NW
CHUNK = 128
NCHUNK = ATOMS_PER_W // CHUNK
TOK_STRIPE = N_TOK // NS
BROWS = N_TOK // 128

TOK_BLK = 1024
_END_SENTINEL = 1 << 30


def _seg_sum_body(feats_hbm, idxflat_hbm, out_sum_hbm, out_st_hbm,
                  out_en_hbm,
                  shared_sum, shared_st, shared_en,
                  idx2d_v, idxb_v, rows_a, rows_b, st_loc, en_loc,
                  zf_v, zi_v, ident_v, sem_a, sem_b, sem_sc):
    c = lax.axis_index("c")
    s = lax.axis_index("s")
    wid = c * NS + s
    a0 = wid * ATOMS_PER_W
    tok0 = s * TOK_STRIPE

    zf16 = jnp.zeros((16,), jnp.float32)
    zi16 = jnp.zeros((16,), jnp.int32)
    for r in range(16):
        for cc in range(8):
            zf_v[r, pl.ds(cc * 16, 16)] = zf16
            zi_v[r, pl.ds(cc * 16, 16)] = zi16
    i16 = lax.iota(jnp.int32, 16)
    ident_v[pl.ds(0, 16)] = i16
    ident_v[pl.ds(16, 16)] = i16 + 16

    for r in range(BROWS):
        for cc in range(8):
            st_loc[r, pl.ds(cc * 16, 16)] = zi16
            en_loc[r, pl.ds(cc * 16, 16)] = zi16
    for k in range(TOK_STRIPE // 16):
        pltpu.sync_copy(zf_v, shared_sum.at[pl.ds(tok0 + 16 * k, 16)])
    pltpu.sync_copy(zi_v.at[pl.ds(0, 2)], shared_st.at[pl.ds(2 * s, 2)])
    pltpu.sync_copy(zi_v.at[pl.ds(0, 2)], shared_en.at[pl.ds(2 * s, 2)])

    idxb_v[pl.ds(0, 16)] = jnp.full((16,), -1, jnp.int32)
    idxb_v[pl.ds(ATOMS_PER_W + 8, 16)] = jnp.full((16,), _END_SENTINEL, jnp.int32)
    pltpu.sync_copy(idxflat_hbm.at[pl.ds(a0, ATOMS_PER_W)],
                    idxb_v.at[pl.ds(8, ATOMS_PER_W)])

    @pl.when(wid > 0)
    def _():
        pltpu.sync_copy(idxflat_hbm.at[pl.ds(a0 - 8, 8)], idxb_v.at[pl.ds(0, 8)])

    @pl.when(wid < NW - 1)
    def _():
        pltpu.sync_copy(idxflat_hbm.at[pl.ds(a0 + ATOMS_PER_W, 8)],
                        idxb_v.at[pl.ds(ATOMS_PER_W + 8, 8)])

    for j in range(NCHUNK):
        pltpu.sync_copy(idxflat_hbm.at[pl.ds(a0 + j * CHUNK, CHUNK)],
                        idx2d_v.at[j])

    plsc.subcore_barrier()

    bufs = (rows_a, rows_b)
    sems = (sem_a, sem_b)

    def _stage(j):
        return pltpu.async_copy(
            feats_hbm.at[pl.ds(a0 + j * CHUNK, CHUNK)], bufs[j % 2], sems[j % 2])

    stage_descs = {0: _stage(0)}
    for j in range(NCHUNK):
        if j + 1 < NCHUNK:
            stage_descs[j + 1] = _stage(j + 1)
        stage_descs.pop(j).wait()
        scat = pltpu.async_copy(
            bufs[j % 2], shared_sum.at[idx2d_v.at[j]], sem_sc, add=True)
        for k in range(CHUNK // 16):
            base = 8 + j * CHUNK + 16 * k
            x = plsc.load_gather(idxb_v, [base + i16])
            xp = plsc.load_gather(idxb_v, [base - 1 + i16])
            xn = plsc.load_gather(idxb_v, [base + 1 + i16])
            gpos = (a0 + j * CHUNK + 16 * k) + i16
            row = lax.shift_right_logical(x, 7)
            col = lax.bitwise_and(x, 127)
            plsc.store_scatter(st_loc, [row, col], gpos, mask=x != xp)
            plsc.store_scatter(en_loc, [row, col], gpos + 1, mask=x != xn)
        scat.wait()
    plsc.subcore_barrier()

    pltpu.sync_copy(st_loc, shared_st.at[ident_v], add=True)
    pltpu.sync_copy(en_loc, shared_en.at[ident_v], add=True)
    plsc.subcore_barrier()

    pltpu.sync_copy(shared_sum.at[pl.ds(tok0, TOK_STRIPE)],
                    out_sum_hbm.at[c, pl.ds(tok0, TOK_STRIPE)])
    pltpu.sync_copy(shared_st.at[pl.ds(2 * s, 2)], out_st_hbm.at[c, pl.ds(2 * s, 2)])
    pltpu.sync_copy(shared_en.at[pl.ds(2 * s, 2)], out_en_hbm.at[c, pl.ds(2 * s, 2)])


def _segment_sum_sc(feats, idx):
    mesh = plsc.VectorSubcoreMesh(core_axis_name="c", subcore_axis_name="s")
    kern = pl.kernel(
        _seg_sum_body,
        out_type=[
            jax.ShapeDtypeStruct((NC, N_TOK, D_FEAT), jnp.float32),
            jax.ShapeDtypeStruct((NC, BROWS, 128), jnp.int32),
            jax.ShapeDtypeStruct((NC, BROWS, 128), jnp.int32),
        ],
        mesh=mesh,
        scratch_types=[
            pltpu.VMEM_SHARED((N_TOK, D_FEAT), jnp.float32),
            pltpu.VMEM_SHARED((BROWS, 128), jnp.int32),
            pltpu.VMEM_SHARED((BROWS, 128), jnp.int32),
            pltpu.VMEM((NCHUNK, CHUNK), jnp.int32),
            pltpu.VMEM((ATOMS_PER_W + 24,), jnp.int32),
            pltpu.VMEM((CHUNK, D_FEAT), jnp.float32),
            pltpu.VMEM((CHUNK, D_FEAT), jnp.float32),
            pltpu.VMEM((BROWS, 128), jnp.int32),
            pltpu.VMEM((BROWS, 128), jnp.int32),
            pltpu.VMEM((16, 128), jnp.float32),
            pltpu.VMEM((16, 128), jnp.int32),
            pltpu.VMEM((32,), jnp.int32),
            pltpu.SemaphoreType.DMA,
            pltpu.SemaphoreType.DMA,
            pltpu.SemaphoreType.DMA,
        ],
        compiler_params=pltpu.CompilerParams(needs_layout_passes=False),
    )
    return kern(feats, idx)


def _combine_body(ps_ref, st_ref, en_ref, rt_ref, table_ref, wa_ref, wp_ref,
                  out_ref):
    seg_sum = ps_ref[0] + ps_ref[1]
    cnt = ((en_ref[0] + en_ref[1]) - (st_ref[0] + st_ref[1])
           ).astype(jnp.float32)
    nrow = TOK_BLK // 128
    pr = lax.broadcasted_iota(jnp.int32, (TOK_BLK, nrow), 0) // 128
    rr = lax.broadcasted_iota(jnp.int32, (TOK_BLK, nrow), 1)
    o1 = (pr == rr).astype(jnp.float32)
    t1 = jnp.dot(o1, cnt, preferred_element_type=jnp.float32,
                 precision=lax.Precision.HIGHEST)
    pc = lax.broadcasted_iota(jnp.int32, (TOK_BLK, 128), 0) % 128
    cc = lax.broadcasted_iota(jnp.int32, (TOK_BLK, 128), 1)
    b0 = t1 * (pc == cc).astype(jnp.float32)
    cnt_bc = jnp.dot(b0, jnp.ones((128, 128), jnp.float32),
                     preferred_element_type=jnp.float32,
                     precision=lax.Precision.HIGHEST)
    mean = seg_sum / jnp.maximum(cnt_bc, 1.0)
    w = jnp.dot(wa_ref[...], wp_ref[...], preferred_element_type=jnp.float32)
    rt = rt_ref[0]
    onehot = (lax.broadcasted_iota(jnp.int32, (N_RESTYPE, TOK_BLK), 0)
              == rt).astype(jnp.float32)
    emb = lax.dot_general(onehot, table_ref[...],
                          dimension_numbers=(((0,), (0,)), ((), ())),
                          preferred_element_type=jnp.float32)
    out_ref[...] = emb + jnp.dot(mean, w, preferred_element_type=jnp.float32)


def _combine_tc(psum, starts, ends, restype, table, w_atom, w_proj):
    n_blk = N_TOK // TOK_BLK
    rt3 = restype.reshape(n_blk, 1, TOK_BLK)
    brows_blk = TOK_BLK // 128
    return pl.pallas_call(
        _combine_body,
        grid=(n_blk,),
        in_specs=[
            pl.BlockSpec((NC, TOK_BLK, D_FEAT), lambda i: (0, i, 0)),
            pl.BlockSpec((NC, brows_blk, 128), lambda i: (0, i, 0)),
            pl.BlockSpec((NC, brows_blk, 128), lambda i: (0, i, 0)),
            pl.BlockSpec((1, 1, TOK_BLK), lambda i: (i, 0, 0)),
            pl.BlockSpec((N_RESTYPE, D_SINGLE), lambda i: (0, 0)),
            pl.BlockSpec((D_FEAT, D_FEAT), lambda i: (0, 0)),
            pl.BlockSpec((D_FEAT, D_SINGLE), lambda i: (0, 0)),
        ],
        out_specs=pl.BlockSpec((TOK_BLK, D_SINGLE), lambda i: (i, 0)),
        out_shape=jax.ShapeDtypeStruct((N_TOK, D_SINGLE), jnp.float32),
    )(psum, starts, ends, rt3, table, w_atom, w_proj)


def kernel(atom_feats, restype, atom_to_token_idx, restype_table, W_atom, W_proj):
    feats = atom_feats.reshape(N_ATOM, D_FEAT)
    idx = atom_to_token_idx.reshape(N_ATOM).astype(jnp.int32)
    rt = restype.reshape(N_TOK).astype(jnp.int32)
    psum, starts, ends = _segment_sum_sc(feats, idx)
    out = _combine_tc(psum, starts, ends, rt, restype_table, W_atom, W_proj)
    return out.reshape(1, N_TOK, D_SINGLE)


# speedup vs baseline: 3.7983x; 1.0215x over previous
import functools

import jax
import jax.numpy as jnp
from jax import lax
from jax.experimental import pallas as pl
from jax.experimental.pallas import tpu as pltpu
from jax.experimental.pallas import tpu_sc as plsc

N_TOK, N_ATOM = 4096, 32768
D_FEAT, D_SINGLE, N_RESTYPE = 128, 384, 32

NC, NS = 2, 16
NW = NC * NS
ATOMS_PER_W = N_ATOM // NW
CHUNK = 128
NCHUNK = ATOMS_PER_W // CHUNK
TOK_STRIPE = N_TOK // NS
BROWS = N_TOK // 128

TOK_BLK = 512
_END_SENTINEL = 1 << 30


def _seg_sum_body(feats_hbm, idx_hbm, idxflat_hbm, out_sum_hbm, out_st_hbm,
                  out_en_hbm,
                  shared_sum, shared_st, shared_en,
                  idx2d_v, idxb_v, rows_a, rows_b, st_loc, en_loc,
                  zf_v, zi_v, ident_v, sem_a, sem_b, sem_sc):
    c = lax.axis_index("c")
    s = lax.axis_index("s")
    wid = c * NS + s
    a0 = wid * ATOMS_PER_W
    tok0 = s * TOK_STRIPE

    zf16 = jnp.zeros((16,), jnp.float32)
    zi16 = jnp.zeros((16,), jnp.int32)
    for r in range(16):
        for cc in range(8):
            zf_v[r, pl.ds(cc * 16, 16)] = zf16
            zi_v[r, pl.ds(cc * 16, 16)] = zi16
    i16 = lax.iota(jnp.int32, 16)
    ident_v[pl.ds(0, 16)] = i16
    ident_v[pl.ds(16, 16)] = i16 + 16

    for r in range(BROWS):
        for cc in range(8):
            st_loc[r, pl.ds(cc * 16, 16)] = zi16
            en_loc[r, pl.ds(cc * 16, 16)] = zi16
    for k in range(TOK_STRIPE // 16):
        pltpu.sync_copy(zf_v, shared_sum.at[pl.ds(tok0 + 16 * k, 16)])
    pltpu.sync_copy(zi_v.at[pl.ds(0, 2)], shared_st.at[pl.ds(2 * s, 2)])
    pltpu.sync_copy(zi_v.at[pl.ds(0, 2)], shared_en.at[pl.ds(2 * s, 2)])

    pltpu.sync_copy(idx_hbm.at[pl.ds(wid * NCHUNK, NCHUNK)], idx2d_v)
    pltpu.sync_copy(idxflat_hbm.at[pl.ds(a0, ATOMS_PER_W + 16)], idxb_v)

    plsc.subcore_barrier()

    bufs = (rows_a, rows_b)
    sems = (sem_a, sem_b)

    def _stage(j):
        return pltpu.async_copy(
            feats_hbm.at[pl.ds(a0 + j * CHUNK, CHUNK)], bufs[j % 2], sems[j % 2])

    stage_descs = {0: _stage(0)}
    for j in range(NCHUNK):
        if j + 1 < NCHUNK:
            stage_descs[j + 1] = _stage(j + 1)
        stage_descs.pop(j).wait()
        scat = pltpu.async_copy(
            bufs[j % 2], shared_sum.at[idx2d_v.at[j]], sem_sc, add=True)
        for k in range(CHUNK // 16):
            base = 8 + j * CHUNK + 16 * k
            x = plsc.load_gather(idxb_v, [base + i16])
            xp = plsc.load_gather(idxb_v, [base - 1 + i16])
            xn = plsc.load_gather(idxb_v, [base + 1 + i16])
            gpos = (a0 + j * CHUNK + 16 * k) + i16
            row = lax.shift_right_logical(x, 7)
            col = lax.bitwise_and(x, 127)
            plsc.store_scatter(st_loc, [row, col], gpos, mask=x != xp)
            plsc.store_scatter(en_loc, [row, col], gpos + 1, mask=x != xn)
        scat.wait()
    plsc.subcore_barrier()

    pltpu.sync_copy(st_loc, shared_st.at[ident_v], add=True)
    pltpu.sync_copy(en_loc, shared_en.at[ident_v], add=True)
    plsc.subcore_barrier()

    pltpu.sync_copy(shared_sum.at[pl.ds(tok0, TOK_STRIPE)],
                    out_sum_hbm.at[c, pl.ds(tok0, TOK_STRIPE)])
    pltpu.sync_copy(shared_st.at[pl.ds(2 * s, 2)], out_st_hbm.at[c, pl.ds(2 * s, 2)])
    pltpu.sync_copy(shared_en.at[pl.ds(2 * s, 2)], out_en_hbm.at[c, pl.ds(2 * s, 2)])


def _segment_sum_sc(feats, idx):
    mesh = plsc.VectorSubcoreMesh(core_axis_name="c", subcore_axis_name="s")
    kern = pl.kernel(
        _seg_sum_body,
        out_type=[
            jax.ShapeDtypeStruct((NC, N_TOK, D_FEAT), jnp.float32),
            jax.ShapeDtypeStruct((NC, BROWS, 128), jnp.int32),
            jax.ShapeDtypeStruct((NC, BROWS, 128), jnp.int32),
        ],
        mesh=mesh,
        scratch_types=[
            pltpu.VMEM_SHARED((N_TOK, D_FEAT), jnp.float32),
            pltpu.VMEM_SHARED((BROWS, 128), jnp.int32),
            pltpu.VMEM_SHARED((BROWS, 128), jnp.int32),
            pltpu.VMEM((NCHUNK, CHUNK), jnp.int32),
            pltpu.VMEM((ATOMS_PER_W + 16,), jnp.int32),
            pltpu.VMEM((CHUNK, D_FEAT), jnp.float32),
            pltpu.VMEM((CHUNK, D_FEAT), jnp.float32),
            pltpu.VMEM((BROWS, 128), jnp.int32),
            pltpu.VMEM((BROWS, 128), jnp.int32),
            pltpu.VMEM((16, 128), jnp.float32),
            pltpu.VMEM((16, 128), jnp.int32),
            pltpu.VMEM((32,), jnp.int32),
            pltpu.SemaphoreType.DMA,
            pltpu.SemaphoreType.DMA,
            pltpu.SemaphoreType.DMA,
        ],
        compiler_params=pltpu.CompilerParams(needs_layout_passes=False),
    )
    idx2d = idx.reshape(NW * NCHUNK, CHUNK)
    idxflat_padded = jnp.concatenate([
        jnp.full((8,), -1, jnp.int32), idx,
        jnp.full((8,), _END_SENTINEL, jnp.int32)])
    return kern(feats, idx2d, idxflat_padded)


def _combine_body(ps_ref, st_ref, en_ref, rt_ref, table_ref, wa_ref, wp_ref,
                  out_ref):
    seg_sum = ps_ref[0] + ps_ref[1]
    cnt = ((en_ref[0] + en_ref[1]) - (st_ref[0] + st_ref[1])
           ).astype(jnp.float32)
    inv = 1.0 / jnp.maximum(cnt, 1.0)
    nrow = BROWS
    p0 = pl.program_id(0) * TOK_BLK
    pr = (p0 + lax.broadcasted_iota(jnp.int32, (TOK_BLK, nrow), 0)) // 128
    rr = lax.broadcasted_iota(jnp.int32, (TOK_BLK, nrow), 1)
    o1 = (pr == rr).astype(jnp.float32)
    t1 = jnp.dot(o1, inv, preferred_element_type=jnp.float32,
                 precision=lax.Precision.HIGHEST)
    pc = (p0 + lax.broadcasted_iota(jnp.int32, (TOK_BLK, 128), 0)) % 128
    cc = lax.broadcasted_iota(jnp.int32, (TOK_BLK, 128), 1)
    b0 = t1 * (pc == cc).astype(jnp.float32)
    inv_bc = jnp.dot(b0, jnp.ones((128, 128), jnp.float32),
                     preferred_element_type=jnp.float32,
                     precision=lax.Precision.HIGHEST)
    mean = seg_sum * inv_bc
    w = jnp.dot(wa_ref[...], wp_ref[...], preferred_element_type=jnp.float32)
    rt = rt_ref[0]
    onehot = (lax.broadcasted_iota(jnp.int32, (N_RESTYPE, TOK_BLK), 0)
              == rt).astype(jnp.float32)
    emb = lax.dot_general(onehot, table_ref[...],
                          dimension_numbers=(((0,), (0,)), ((), ())),
                          preferred_element_type=jnp.float32)
    out_ref[0] = emb + jnp.dot(mean, w, preferred_element_type=jnp.float32)


def _combine_tc(psum, starts, ends, restype, table, w_atom, w_proj):
    n_blk = N_TOK // TOK_BLK
    rt3 = restype.reshape(n_blk, 1, TOK_BLK)
    brows_blk = TOK_BLK // 128
    return pl.pallas_call(
        _combine_body,
        grid=(n_blk,),
        in_specs=[
            pl.BlockSpec((NC, TOK_BLK, D_FEAT), lambda i: (0, i, 0)),
            pl.BlockSpec((NC, BROWS, 128), lambda i: (0, 0, 0)),
            pl.BlockSpec((NC, BROWS, 128), lambda i: (0, 0, 0)),
            pl.BlockSpec((1, 1, TOK_BLK), lambda i: (i, 0, 0)),
            pl.BlockSpec((N_RESTYPE, D_SINGLE), lambda i: (0, 0)),
            pl.BlockSpec((D_FEAT, D_FEAT), lambda i: (0, 0)),
            pl.BlockSpec((D_FEAT, D_SINGLE), lambda i: (0, 0)),
        ],
        out_specs=pl.BlockSpec((1, TOK_BLK, D_SINGLE), lambda i: (0, i, 0)),
        out_shape=jax.ShapeDtypeStruct((1, N_TOK, D_SINGLE), jnp.float32),
    )(psum, starts, ends, rt3, table, w_atom, w_proj)


def kernel(atom_feats, restype, atom_to_token_idx, restype_table, W_atom, W_proj):
    feats = atom_feats.reshape(N_ATOM, D_FEAT)
    idx = atom_to_token_idx.reshape(N_ATOM).astype(jnp.int32)
    rt = restype.reshape(N_TOK).astype(jnp.int32)
    psum, starts, ends = _segment_sum_sc(feats, idx)
    return _combine_tc(psum, starts, ends, rt, restype_table, W_atom, W_proj)


# speedup vs baseline: 3.8362x; 1.0100x over previous
import functools

import jax
import jax.numpy as jnp
from jax import lax
from jax.experimental import pallas as pl
from jax.experimental.pallas import tpu as pltpu
from jax.experimental.pallas import tpu_sc as plsc

N_TOK, N_ATOM = 4096, 32768
D_FEAT, D_SINGLE, N_RESTYPE = 128, 384, 32

NC, NS = 2, 16
NW = NC * NS
ATOMS_PER_W = N_ATOM // NW
CHUNK = 128
NCHUNK = ATOMS_PER_W // CHUNK
TOK_STRIPE = N_TOK // NS
BROWS = N_TOK // 128

TOK_BLK = 1024
_END_SENTINEL = 1 << 30


def _seg_sum_body(feats_hbm, idx_hbm, out_sum_hbm, out_st_hbm,
                  out_en_hbm,
                  shared_sum, shared_st, shared_en,
                  idx2d_v, rows_a, rows_b, rows_c, st_loc, en_loc,
                  zf_v, zi_v, ident_v, sem_a, sem_b, sem_sc, sem_sd):
    c = lax.axis_index("c")
    s = lax.axis_index("s")
    wid = c * NS + s
    a0 = wid * ATOMS_PER_W
    tok0 = s * TOK_STRIPE

    zf16 = jnp.zeros((16,), jnp.float32)
    zi16 = jnp.zeros((16,), jnp.int32)
    for r in range(16):
        for cc in range(8):
            zf_v[r, pl.ds(cc * 16, 16)] = zf16
            zi_v[r, pl.ds(cc * 16, 16)] = zi16
    i16 = lax.iota(jnp.int32, 16)
    ident_v[pl.ds(0, 16)] = i16
    ident_v[pl.ds(16, 16)] = i16 + 16

    idx2d_v[NCHUNK, pl.ds(CHUNK - 16, 16)] = jnp.full((16,), -1, jnp.int32)
    idx2d_v[NCHUNK + 1, pl.ds(0, 16)] = jnp.full((16,), _END_SENTINEL, jnp.int32)

    pltpu.sync_copy(idx_hbm.at[pl.ds(wid * NCHUNK, NCHUNK)],
                    idx2d_v.at[pl.ds(0, NCHUNK)])

    @pl.when(wid > 0)
    def _():
        pltpu.sync_copy(idx_hbm.at[pl.ds(wid * NCHUNK - 1, 1)],
                        idx2d_v.at[pl.ds(NCHUNK, 1)])

    @pl.when(wid < NW - 1)
    def _():
        pltpu.sync_copy(idx_hbm.at[pl.ds((wid + 1) * NCHUNK, 1)],
                        idx2d_v.at[pl.ds(NCHUNK + 1, 1)])

    for k in range(TOK_STRIPE // 16):
        pltpu.sync_copy(zf_v, shared_sum.at[pl.ds(tok0 + 16 * k, 16)])
    pltpu.sync_copy(zi_v.at[pl.ds(0, 2)], shared_st.at[pl.ds(2 * s, 2)])
    pltpu.sync_copy(zi_v.at[pl.ds(0, 2)], shared_en.at[pl.ds(2 * s, 2)])

    for r in range(BROWS):
        for cc in range(8):
            st_loc[r, pl.ds(cc * 16, 16)] = zi16
            en_loc[r, pl.ds(cc * 16, 16)] = zi16

    plsc.subcore_barrier()

    bufs = (rows_a, rows_b)
    sems = (sem_a, sem_b)

    def _stage(j):
        return pltpu.async_copy(
            feats_hbm.at[pl.ds(a0 + j * CHUNK, CHUNK)], bufs[j % 2], sems[j % 2])

    stage_descs = {0: _stage(0)}
    for j in range(NCHUNK):
        if j + 1 < NCHUNK:
            stage_descs[j + 1] = _stage(j + 1)
        stage_descs.pop(j).wait()
        scat = pltpu.async_copy(
            bufs[j % 2], shared_sum.at[idx2d_v.at[j]], sem_sc, add=True)
        for k in range(CHUNK // 16):
            x = idx2d_v[j, pl.ds(16 * k, 16)]
            p = j * CHUNK + 16 * k + i16
            pm1 = p - 1
            pp1 = p + 1
            if j == 0 and k == 0:
                xp_row = jnp.where(i16 == 0, NCHUNK, lax.shift_right_logical(pm1, 7))
                xp_col = jnp.where(i16 == 0, CHUNK - 1, lax.bitwise_and(pm1, 127))
            else:
                xp_row = lax.shift_right_logical(pm1, 7)
                xp_col = lax.bitwise_and(pm1, 127)
            if j == NCHUNK - 1 and k == CHUNK // 16 - 1:
                xn_row = jnp.where(i16 == 15, NCHUNK + 1, lax.shift_right_logical(pp1, 7))
                xn_col = jnp.where(i16 == 15, 0, lax.bitwise_and(pp1, 127))
            else:
                xn_row = lax.shift_right_logical(pp1, 7)
                xn_col = lax.bitwise_and(pp1, 127)
            xp = plsc.load_gather(idx2d_v, [xp_row, xp_col])
            xn = plsc.load_gather(idx2d_v, [xn_row, xn_col])
            gpos = a0 + p
            row = lax.shift_right_logical(x, 7)
            col = lax.bitwise_and(x, 127)
            plsc.store_scatter(st_loc, [row, col], gpos, mask=x != xp)
            plsc.store_scatter(en_loc, [row, col], gpos + 1, mask=x != xn)
        scat.wait()
    plsc.subcore_barrier()

    pltpu.sync_copy(st_loc, shared_st.at[ident_v], add=True)
    pltpu.sync_copy(en_loc, shared_en.at[ident_v], add=True)
    plsc.subcore_barrier()

    pltpu.sync_copy(shared_sum.at[pl.ds(tok0, TOK_STRIPE)],
                    out_sum_hbm.at[c, pl.ds(tok0, TOK_STRIPE)])
    pltpu.sync_copy(shared_st.at[pl.ds(2 * s, 2)], out_st_hbm.at[c, pl.ds(2 * s, 2)])
    pltpu.sync_copy(shared_en.at[pl.ds(2 * s, 2)], out_en_hbm.at[c, pl.ds(2 * s, 2)])


def _segment_sum_sc(feats, idx):
    mesh = plsc.VectorSubcoreMesh(core_axis_name="c", subcore_axis_name="s")
    kern = pl.kernel(
        _seg_sum_body,
        out_type=[
            jax.ShapeDtypeStruct((NC, N_TOK, D_FEAT), jnp.float32),
            jax.ShapeDtypeStruct((NC, BROWS, 128), jnp.int32),
            jax.ShapeDtypeStruct((NC, BROWS, 128), jnp.int32),
        ],
        mesh=mesh,
        scratch_types=[
            pltpu.VMEM_SHARED((N_TOK, D_FEAT), jnp.float32),
            pltpu.VMEM_SHARED((BROWS, 128), jnp.int32),
            pltpu.VMEM_SHARED((BROWS, 128), jnp.int32),
            pltpu.VMEM((NCHUNK + 2, CHUNK), jnp.int32),
            pltpu.VMEM((CHUNK, D_FEAT), jnp.float32),
            pltpu.VMEM((CHUNK, D_FEAT), jnp.float32),
            pltpu.VMEM((CHUNK, D_FEAT), jnp.float32),
            pltpu.VMEM((BROWS, 128), jnp.int32),
            pltpu.VMEM((BROWS, 128), jnp.int32),
            pltpu.VMEM((16, 128), jnp.float32),
            pltpu.VMEM((16, 128), jnp.int32),
            pltpu.VMEM((32,), jnp.int32),
            pltpu.SemaphoreType.DMA,
            pltpu.SemaphoreType.DMA,
            pltpu.SemaphoreType.DMA,
            pltpu.SemaphoreType.DMA,
        ],
        compiler_params=pltpu.CompilerParams(needs_layout_passes=False),
    )
    idx2d = idx.reshape(NW * NCHUNK, CHUNK)
    return kern(feats, idx2d)


def _combine_body(ps_ref, st_ref, en_ref, rt_ref, table_ref, wa_ref, wp_ref,
                  out_ref):
    seg_sum = ps_ref[0] + ps_ref[1]
    cnt = ((en_ref[0] + en_ref[1]) - (st_ref[0] + st_ref[1])
           ).astype(jnp.float32)
    inv = 1.0 / jnp.maximum(cnt, 1.0)
    nrow = BROWS
    p0 = pl.program_id(0) * TOK_BLK
    pr = (p0 + lax.broadcasted_iota(jnp.int32, (TOK_BLK, nrow), 0)) // 128
    rr = lax.broadcasted_iota(jnp.int32, (TOK_BLK, nrow), 1)
    o1 = (pr == rr).astype(jnp.float32)
    t1 = jnp.dot(o1, inv, preferred_element_type=jnp.float32,
                 precision=lax.Precision.HIGHEST)
    pc = (p0 + lax.broadcasted_iota(jnp.int32, (TOK_BLK, 128), 0)) % 128
    cc = lax.broadcasted_iota(jnp.int32, (TOK_BLK, 128), 1)
    b0 = t1 * (pc == cc).astype(jnp.float32)
    inv_bc = jnp.dot(b0, jnp.ones((128, 128), jnp.float32),
                     preferred_element_type=jnp.float32,
                     precision=lax.Precision.HIGHEST)
    mean = seg_sum * inv_bc
    w = jnp.dot(wa_ref[...], wp_ref[...], preferred_element_type=jnp.float32)
    rt = rt_ref[0]
    onehot = (lax.broadcasted_iota(jnp.int32, (N_RESTYPE, TOK_BLK), 0)
              == rt).astype(jnp.float32)
    emb = lax.dot_general(onehot, table_ref[...],
                          dimension_numbers=(((0,), (0,)), ((), ())),
                          preferred_element_type=jnp.float32)
    out_ref[0] = emb + jnp.dot(mean, w, preferred_element_type=jnp.float32)


def _combine_tc(psum, starts, ends, restype, table, w_atom, w_proj):
    n_blk = N_TOK // TOK_BLK
    rt3 = restype.reshape(n_blk, 1, TOK_BLK)
    brows_blk = TOK_BLK // 128
    return pl.pallas_call(
        _combine_body,
        grid=(n_blk,),
        in_specs=[
            pl.BlockSpec((NC, TOK_BLK, D_FEAT), lambda i: (0, i, 0)),
            pl.BlockSpec((NC, BROWS, 128), lambda i: (0, 0, 0)),
            pl.BlockSpec((NC, BROWS, 128), lambda i: (0, 0, 0)),
            pl.BlockSpec((1, 1, TOK_BLK), lambda i: (i, 0, 0)),
            pl.BlockSpec((N_RESTYPE, D_SINGLE), lambda i: (0, 0)),
            pl.BlockSpec((D_FEAT, D_FEAT), lambda i: (0, 0)),
            pl.BlockSpec((D_FEAT, D_SINGLE), lambda i: (0, 0)),
        ],
        out_specs=pl.BlockSpec((1, TOK_BLK, D_SINGLE), lambda i: (0, i, 0)),
        out_shape=jax.ShapeDtypeStruct((1, N_TOK, D_SINGLE), jnp.float32),
    )(psum, starts, ends, rt3, table, w_atom, w_proj)


def kernel(atom_feats, restype, atom_to_token_idx, restype_table, W_atom, W_proj):
    feats = atom_feats.reshape(N_ATOM, D_FEAT)
    idx = atom_to_token_idx.reshape(N_ATOM).astype(jnp.int32)
    rt = restype.reshape(N_TOK).astype(jnp.int32)
    psum, starts, ends = _segment_sum_sc(feats, idx)
    return _combine_tc(psum, starts, ends, rt, restype_table, W_atom, W_proj)


# speedup vs baseline: 3.8424x; 1.0016x over previous
import functools

import jax
import jax.numpy as jnp
from jax import lax
from jax.experimental import pallas as pl
from jax.experimental.pallas import tpu as pltpu
from jax.experimental.pallas import tpu_sc as plsc

N_TOK, N_ATOM = 4096, 32768
D_FEAT, D_SINGLE, N_RESTYPE = 128, 384, 32

NC, NS = 2, 16
NW = NC * NS
ATOMS_PER_W = N_ATOM // NW
CHUNK = 128
NCHUNK = ATOMS_PER_W // CHUNK
TOK_STRIPE = N_TOK // NS
BROWS = N_TOK // 128

TOK_BLK = 1024
_END_SENTINEL = 1 << 30


def _seg_sum_body(feats_hbm, idx_hbm, out_sum_hbm, out_st_hbm,
                  out_en_hbm,
                  shared_sum, shared_st, shared_en,
                  idx2d_v, rows_a, rows_b, rows_c, st_loc, en_loc,
                  zf_v, zi_v, ident_v, sem_a, sem_b, sem_sc, sem_sd):
    c = lax.axis_index("c")
    s = lax.axis_index("s")
    wid = c * NS + s
    a0 = wid * ATOMS_PER_W
    tok0 = s * TOK_STRIPE

    zf16 = jnp.zeros((16,), jnp.float32)
    zi16 = jnp.zeros((16,), jnp.int32)
    for r in range(16):
        for cc in range(8):
            zf_v[r, pl.ds(cc * 16, 16)] = zf16
            zi_v[r, pl.ds(cc * 16, 16)] = zi16
    i16 = lax.iota(jnp.int32, 16)
    ident_v[pl.ds(0, 16)] = i16
    ident_v[pl.ds(16, 16)] = i16 + 16

    idx2d_v[NCHUNK, pl.ds(CHUNK - 16, 16)] = jnp.full((16,), -1, jnp.int32)
    idx2d_v[NCHUNK + 1, pl.ds(0, 16)] = jnp.full((16,), _END_SENTINEL, jnp.int32)

    pltpu.sync_copy(idx_hbm.at[pl.ds(wid * NCHUNK, NCHUNK)],
                    idx2d_v.at[pl.ds(0, NCHUNK)])

    @pl.when(wid > 0)
    def _():
        pltpu.sync_copy(idx_hbm.at[pl.ds(wid * NCHUNK - 1, 1)],
                        idx2d_v.at[pl.ds(NCHUNK, 1)])

    @pl.when(wid < NW - 1)
    def _():
        pltpu.sync_copy(idx_hbm.at[pl.ds((wid + 1) * NCHUNK, 1)],
                        idx2d_v.at[pl.ds(NCHUNK + 1, 1)])

    for k in range(TOK_STRIPE // 16):
        pltpu.sync_copy(zf_v, shared_sum.at[pl.ds(tok0 + 16 * k, 16)])
    pltpu.sync_copy(zi_v.at[pl.ds(0, 2)], shared_st.at[pl.ds(2 * s, 2)])
    pltpu.sync_copy(zi_v.at[pl.ds(0, 2)], shared_en.at[pl.ds(2 * s, 2)])

    for r in range(BROWS):
        for cc in range(8):
            st_loc[r, pl.ds(cc * 16, 16)] = zi16
            en_loc[r, pl.ds(cc * 16, 16)] = zi16

    plsc.subcore_barrier()

    bufs = (rows_a, rows_b, rows_c)
    sems = (sem_a, sem_b)
    ssems = (sem_sc, sem_sd)

    def _stage(j):
        return pltpu.async_copy(
            feats_hbm.at[pl.ds(a0 + j * CHUNK, CHUNK)], bufs[j % 3], sems[j % 2])

    stage_descs = {0: _stage(0)}
    scat_descs = {}
    for j in range(NCHUNK):
        if j >= 2:
            scat_descs.pop(j - 2).wait()
        if j + 1 < NCHUNK:
            stage_descs[j + 1] = _stage(j + 1)
        stage_descs.pop(j).wait()
        scat_descs[j] = pltpu.async_copy(
            bufs[j % 3], shared_sum.at[idx2d_v.at[j]], ssems[j % 2], add=True)
        for k in range(CHUNK // 16):
            x = idx2d_v[j, pl.ds(16 * k, 16)]
            p = j * CHUNK + 16 * k + i16
            pm1 = p - 1
            pp1 = p + 1
            if j == 0 and k == 0:
                xp_row = jnp.where(i16 == 0, NCHUNK, lax.shift_right_logical(pm1, 7))
                xp_col = jnp.where(i16 == 0, CHUNK - 1, lax.bitwise_and(pm1, 127))
            else:
                xp_row = lax.shift_right_logical(pm1, 7)
                xp_col = lax.bitwise_and(pm1, 127)
            if j == NCHUNK - 1 and k == CHUNK // 16 - 1:
                xn_row = jnp.where(i16 == 15, NCHUNK + 1, lax.shift_right_logical(pp1, 7))
                xn_col = jnp.where(i16 == 15, 0, lax.bitwise_and(pp1, 127))
            else:
                xn_row = lax.shift_right_logical(pp1, 7)
                xn_col = lax.bitwise_and(pp1, 127)
            xp = plsc.load_gather(idx2d_v, [xp_row, xp_col])
            xn = plsc.load_gather(idx2d_v, [xn_row, xn_col])
            gpos = a0 + p
            row = lax.shift_right_logical(x, 7)
            col = lax.bitwise_and(x, 127)
            plsc.store_scatter(st_loc, [row, col], gpos, mask=x != xp)
            plsc.store_scatter(en_loc, [row, col], gpos + 1, mask=x != xn)
    scat_descs.pop(NCHUNK - 2).wait()
    scat_descs.pop(NCHUNK - 1).wait()
    plsc.subcore_barrier()

    pltpu.sync_copy(st_loc, shared_st.at[ident_v], add=True)
    pltpu.sync_copy(en_loc, shared_en.at[ident_v], add=True)
    plsc.subcore_barrier()

    pltpu.sync_copy(shared_sum.at[pl.ds(tok0, TOK_STRIPE)],
                    out_sum_hbm.at[c, pl.ds(tok0, TOK_STRIPE)])
    pltpu.sync_copy(shared_st.at[pl.ds(2 * s, 2)], out_st_hbm.at[c, pl.ds(2 * s, 2)])
    pltpu.sync_copy(shared_en.at[pl.ds(2 * s, 2)], out_en_hbm.at[c, pl.ds(2 * s, 2)])


def _segment_sum_sc(feats, idx):
    mesh = plsc.VectorSubcoreMesh(core_axis_name="c", subcore_axis_name="s")
    kern = pl.kernel(
        _seg_sum_body,
        out_type=[
            jax.ShapeDtypeStruct((NC, N_TOK, D_FEAT), jnp.float32),
            jax.ShapeDtypeStruct((NC, BROWS, 128), jnp.int32),
            jax.ShapeDtypeStruct((NC, BROWS, 128), jnp.int32),
        ],
        mesh=mesh,
        scratch_types=[
            pltpu.VMEM_SHARED((N_TOK, D_FEAT), jnp.float32),
            pltpu.VMEM_SHARED((BROWS, 128), jnp.int32),
            pltpu.VMEM_SHARED((BROWS, 128), jnp.int32),
            pltpu.VMEM((NCHUNK + 2, CHUNK), jnp.int32),
            pltpu.VMEM((CHUNK, D_FEAT), jnp.float32),
            pltpu.VMEM((CHUNK, D_FEAT), jnp.float32),
            pltpu.VMEM((CHUNK, D_FEAT), jnp.float32),
            pltpu.VMEM((BROWS, 128), jnp.int32),
            pltpu.VMEM((BROWS, 128), jnp.int32),
            pltpu.VMEM((16, 128), jnp.float32),
            pltpu.VMEM((16, 128), jnp.int32),
            pltpu.VMEM((32,), jnp.int32),
            pltpu.SemaphoreType.DMA,
            pltpu.SemaphoreType.DMA,
            pltpu.SemaphoreType.DMA,
            pltpu.SemaphoreType.DMA,
        ],
        compiler_params=pltpu.CompilerParams(needs_layout_passes=False),
    )
    idx2d = idx.reshape(NW * NCHUNK, CHUNK)
    return kern(feats, idx2d)


def _combine_body(ps_ref, st_ref, en_ref, rt_ref, table_ref, wa_ref, wp_ref,
                  out_ref):
    seg_sum = ps_ref[0] + ps_ref[1]
    cnt = ((en_ref[0] + en_ref[1]) - (st_ref[0] + st_ref[1])
           ).astype(jnp.float32)
    inv = 1.0 / jnp.maximum(cnt, 1.0)
    nrow = BROWS
    p0 = pl.program_id(0) * TOK_BLK
    pr = (p0 + lax.broadcasted_iota(jnp.int32, (TOK_BLK, nrow), 0)) // 128
    rr = lax.broadcasted_iota(jnp.int32, (TOK_BLK, nrow), 1)
    o1 = (pr == rr).astype(jnp.float32)
    t1 = jnp.dot(o1, inv, preferred_element_type=jnp.float32,
                 precision=lax.Precision.HIGHEST)
    pc = (p0 + lax.broadcasted_iota(jnp.int32, (TOK_BLK, 128), 0)) % 128
    cc = lax.broadcasted_iota(jnp.int32, (TOK_BLK, 128), 1)
    b0 = t1 * (pc == cc).astype(jnp.float32)
    inv_bc = jnp.dot(b0, jnp.ones((128, 128), jnp.float32),
                     preferred_element_type=jnp.float32,
                     precision=lax.Precision.HIGHEST)
    mean = seg_sum * inv_bc
    w = jnp.dot(wa_ref[...], wp_ref[...], preferred_element_type=jnp.float32)
    rt = rt_ref[0]
    onehot = (lax.broadcasted_iota(jnp.int32, (N_RESTYPE, TOK_BLK), 0)
              == rt).astype(jnp.float32)
    emb = lax.dot_general(onehot, table_ref[...],
                          dimension_numbers=(((0,), (0,)), ((), ())),
                          preferred_element_type=jnp.float32)
    out_ref[0] = emb + jnp.dot(mean, w, preferred_element_type=jnp.float32)


def _combine_tc(psum, starts, ends, restype, table, w_atom, w_proj):
    n_blk = N_TOK // TOK_BLK
    rt3 = restype.reshape(n_blk, 1, TOK_BLK)
    brows_blk = TOK_BLK // 128
    return pl.pallas_call(
        _combine_body,
        grid=(n_blk,),
        in_specs=[
            pl.BlockSpec((NC, TOK_BLK, D_FEAT), lambda i: (0, i, 0)),
            pl.BlockSpec((NC, BROWS, 128), lambda i: (0, 0, 0)),
            pl.BlockSpec((NC, BROWS, 128), lambda i: (0, 0, 0)),
            pl.BlockSpec((1, 1, TOK_BLK), lambda i: (i, 0, 0)),
            pl.BlockSpec((N_RESTYPE, D_SINGLE), lambda i: (0, 0)),
            pl.BlockSpec((D_FEAT, D_FEAT), lambda i: (0, 0)),
            pl.BlockSpec((D_FEAT, D_SINGLE), lambda i: (0, 0)),
        ],
        out_specs=pl.BlockSpec((1, TOK_BLK, D_SINGLE), lambda i: (0, i, 0)),
        out_shape=jax.ShapeDtypeStruct((1, N_TOK, D_SINGLE), jnp.float32),
    )(psum, starts, ends, rt3, table, w_atom, w_proj)


def kernel(atom_feats, restype, atom_to_token_idx, restype_table, W_atom, W_proj):
    feats = atom_feats.reshape(N_ATOM, D_FEAT)
    idx = atom_to_token_idx.reshape(N_ATOM).astype(jnp.int32)
    rt = restype.reshape(N_TOK).astype(jnp.int32)
    psum, starts, ends = _segment_sum_sc(feats, idx)
    return _combine_tc(psum, starts, ends, rt, restype_table, W_atom, W_proj)
